# baseline (speedup 1.0000x reference)
import functools

import jax
import jax.numpy as jnp
import numpy as np
from jax import lax
from jax.experimental import pallas as pl
from jax.experimental.pallas import tpu as pltpu
from jax.experimental.pallas import tpu_sc as plsc

NUM_EMBEDDINGS = 100
EMBED_DIM = 50
PADDED_DIM = 64
BATCH = 16384

NUM_CORES = 2
NUM_SUBCORES = 16
NUM_WORKERS = NUM_CORES * NUM_SUBCORES
ROWS_PER_WORKER = BATCH // NUM_WORKERS
OUT_PER_WORKER = ROWS_PER_WORKER * EMBED_DIM
LANES = 16

GROUP_ROWS = 16
GROUP_F32 = GROUP_ROWS * EMBED_DIM
VREGS_PER_GROUP = GROUP_F32 // LANES
GROUPS_PER_WORKER = ROWS_PER_WORKER // GROUP_ROWS

_P = np.arange(GROUP_F32, dtype=np.int32)
_ROFF = _P // EMBED_DIM
_COFF = _P % EMBED_DIM


def kernel(table, indices):
    mesh = plsc.VectorSubcoreMesh(core_axis_name="c", subcore_axis_name="s")
    table_flat = jnp.pad(
        table, ((0, 0), (0, PADDED_DIM - EMBED_DIM))).reshape(-1)
    idx2 = indices.reshape(NUM_WORKERS, ROWS_PER_WORKER)
    roff = jnp.asarray(_ROFF)
    coff = jnp.asarray(_COFF)

    @functools.partial(
        pl.kernel,
        mesh=mesh,
        out_type=jax.ShapeDtypeStruct((BATCH * EMBED_DIM,), jnp.float32),
        scratch_types=[
            pltpu.VMEM((NUM_EMBEDDINGS * PADDED_DIM,), jnp.float32),
            pltpu.VMEM((ROWS_PER_WORKER,), jnp.int32),
            pltpu.VMEM((GROUP_F32,), jnp.int32),
            pltpu.VMEM((GROUP_F32,), jnp.int32),
            pltpu.VMEM((OUT_PER_WORKER,), jnp.float32),
            pltpu.SemaphoreType.DMA,
        ],
        compiler_params=pltpu.CompilerParams(needs_layout_passes=False),
    )
    def emb_kernel(table_hbm, idx_hbm, roff_hbm, coff_hbm, out_hbm,
                   tab_v, idx_v, roff_v, coff_v, out_v, sem):
        wid = lax.axis_index("s") * NUM_CORES + lax.axis_index("c")
        pltpu.async_copy(table_hbm, tab_v, sem)
        pltpu.async_copy(idx_hbm.at[wid], idx_v, sem)
        pltpu.async_copy(roff_hbm, roff_v, sem)
        pltpu.async_copy(coff_hbm, coff_v, sem)
        pltpu.make_async_copy(table_hbm, tab_v, sem).wait()
        pltpu.make_async_copy(idx_hbm.at[wid], idx_v, sem).wait()
        pltpu.make_async_copy(roff_hbm, roff_v, sem).wait()
        pltpu.make_async_copy(coff_hbm, coff_v, sem).wait()

        @pl.loop(0, GROUPS_PER_WORKER)
        def _(g):
            row_base = g * GROUP_ROWS
            out_base = pl.multiple_of(g * GROUP_F32, LANES)
            for k in range(VREGS_PER_GROUP):
                rv = roff_v[pl.ds(k * LANES, LANES)]
                cv = coff_v[pl.ds(k * LANES, LANES)]
                iv = plsc.load_gather(idx_v, [row_base + rv])
                vals = plsc.load_gather(tab_v, [(iv << 6) + cv])
                out_v[pl.ds(out_base + k * LANES, LANES)] = vals

        pltpu.sync_copy(out_v, out_hbm.at[pl.ds(wid * OUT_PER_WORKER,
                                                OUT_PER_WORKER)])

    out = emb_kernel(table_flat, idx2, roff, coff)
    return out.reshape(BATCH, EMBED_DIM)


# speedup vs baseline: 1.2821x; 1.2821x over previous
import functools

import jax
import jax.numpy as jnp
import numpy as np
from jax import lax
from jax.experimental import pallas as pl
from jax.experimental.pallas import tpu as pltpu
from jax.experimental.pallas import tpu_sc as plsc

NUM_EMBEDDINGS = 100
EMBED_DIM = 50
PADDED_DIM = 64
BATCH = 16384

NUM_CORES = 2
NUM_SUBCORES = 16
NUM_WORKERS = NUM_CORES * NUM_SUBCORES
ROWS_PER_WORKER = BATCH // NUM_WORKERS
OUT_PER_WORKER = ROWS_PER_WORKER * EMBED_DIM
LANES = 16

GROUP_ROWS = 16
GROUP_F32 = GROUP_ROWS * EMBED_DIM
VREGS_PER_GROUP = GROUP_F32 // LANES
GROUPS_PER_WORKER = ROWS_PER_WORKER // GROUP_ROWS

_P = np.arange(GROUP_F32, dtype=np.int32)
_ROFF = _P // EMBED_DIM
_COFF = _P % EMBED_DIM


def kernel(table, indices):
    mesh = plsc.VectorSubcoreMesh(core_axis_name="c", subcore_axis_name="s")
    table_flat = jnp.pad(
        table, ((0, 0), (0, PADDED_DIM - EMBED_DIM))).reshape(-1)
    idx2 = indices.reshape(NUM_WORKERS, ROWS_PER_WORKER)
    roff = jnp.asarray(_ROFF)
    coff = jnp.asarray(_COFF)

    @functools.partial(
        pl.kernel,
        mesh=mesh,
        out_type=jax.ShapeDtypeStruct((BATCH * EMBED_DIM,), jnp.float32),
        scratch_types=[
            pltpu.VMEM((NUM_EMBEDDINGS * PADDED_DIM,), jnp.float32),
            pltpu.VMEM((ROWS_PER_WORKER,), jnp.int32),
            pltpu.VMEM((GROUP_F32,), jnp.int32),
            pltpu.VMEM((GROUP_F32,), jnp.int32),
            pltpu.VMEM((OUT_PER_WORKER,), jnp.float32),
            pltpu.SemaphoreType.DMA,
        ],
        compiler_params=pltpu.CompilerParams(needs_layout_passes=False),
    )
    def emb_kernel(table_hbm, idx_hbm, roff_hbm, coff_hbm, out_hbm,
                   tab_v, idx_v, roff_v, coff_v, out_v, sem):
        wid = lax.axis_index("s") * NUM_CORES + lax.axis_index("c")
        pltpu.async_copy(table_hbm, tab_v, sem)
        pltpu.async_copy(idx_hbm.at[wid], idx_v, sem)
        pltpu.async_copy(roff_hbm, roff_v, sem)
        pltpu.async_copy(coff_hbm, coff_v, sem)
        pltpu.make_async_copy(table_hbm, tab_v, sem).wait()
        pltpu.make_async_copy(idx_hbm.at[wid], idx_v, sem).wait()
        pltpu.make_async_copy(roff_hbm, roff_v, sem).wait()
        pltpu.make_async_copy(coff_hbm, coff_v, sem).wait()

        @plsc.parallel_loop(0, GROUPS_PER_WORKER, unroll=2)
        def _(g):
            row_base = g * GROUP_ROWS
            out_base = pl.multiple_of(g * GROUP_F32, LANES)
            for k in range(VREGS_PER_GROUP):
                rv = roff_v[pl.ds(k * LANES, LANES)]
                cv = coff_v[pl.ds(k * LANES, LANES)]
                iv = plsc.load_gather(idx_v, [row_base + rv])
                vals = plsc.load_gather(tab_v, [(iv << 6) + cv])
                out_v[pl.ds(out_base + k * LANES, LANES)] = vals

        pltpu.sync_copy(out_v, out_hbm.at[pl.ds(wid * OUT_PER_WORKER,
                                                OUT_PER_WORKER)])

    out = emb_kernel(table_flat, idx2, roff, coff)
    return out.reshape(BATCH, EMBED_DIM)
